# Initial kernel scaffold; baseline (speedup 1.0000x reference)
import jax
import jax.numpy as jnp
from jax.experimental import pallas as pl


def kernel(x, edge_index, batch, g1w1, g1b1, g1w2, g1b2, bn1g, bn1b, g2w1, g2b1, g2w2, g2b2, bn2g, bn2b, g3w1, g3b1, g3w2, g3b2, bn3g, bn3b, g4w1, g4b1, g4w2, g4b2, bn4g, bn4b, gatw, gatas, gatad, gatb, bn5g, bn5b, poolw, poolb, fc1w, fc1b, fc2w, fc2b):
    raise NotImplementedError("write your pallas kernel here")


# speedup vs baseline: 9.7072x; 9.7072x over previous
import functools

import jax
import jax.numpy as jnp
from jax import lax
from jax.experimental import pallas as pl
from jax.experimental.pallas import tpu as pltpu
from jax.experimental.pallas import tpu_sc as plsc

N, E, G = 10000, 320000, 64
D, H, HEADS = 128, 128, 8
F32 = jnp.float32

NSC, NTILE = 2, 16
NW = NSC * NTILE
K = 80
CHUNKS = E // (NW * K)
ROWS_A = 624
TAIL0, TAILN = NTILE * ROWS_A, N - NTILE * ROWS_A

_sc_mesh = plsc.VectorSubcoreMesh(core_axis_name="c", subcore_axis_name="s")


def _zero_acc(s, zrow, acc):
    pltpu.sync_copy(zrow, acc.at[pl.ds(s * ROWS_A, ROWS_A)])

    @pl.when(s == NTILE - 1)
    def _():
        pltpu.sync_copy(zrow.at[pl.ds(0, TAILN)], acc.at[pl.ds(TAIL0, TAILN)])


def _write_acc(s, acc, out_ref):
    pltpu.sync_copy(acc.at[pl.ds(s * ROWS_A, ROWS_A)],
                    out_ref.at[pl.ds(s * ROWS_A, ROWS_A)])

    @pl.when(s == NTILE - 1)
    def _():
        pltpu.sync_copy(acc.at[pl.ds(TAIL0, TAILN)],
                        out_ref.at[pl.ds(TAIL0, TAILN)])


def _make_sc_segsum(nb):
    @functools.partial(
        pl.kernel,
        out_type=jax.ShapeDtypeStruct((NSC, nb, N, 128), F32),
        mesh=_sc_mesh,
        scratch_types=[
            pltpu.VMEM((CHUNKS, K), jnp.int32),
            pltpu.VMEM((CHUNKS, K), jnp.int32),
            pltpu.VMEM((K, 128), F32),
            pltpu.VMEM_SHARED((N, 128), F32),
            pltpu.SemaphoreType.DMA,
        ],
        name=f"sc_segsum_nb{nb}",
    )
    def seg(xp, srcp, dstp, zrow, out, idx_s, idx_d, rows, acc, sem):
        c = lax.axis_index("c")
        s = lax.axis_index("s")
        wid = c * NTILE + s
        pltpu.sync_copy(srcp.at[wid], idx_s)
        pltpu.sync_copy(dstp.at[wid], idx_d)
        for fb in range(nb):
            _zero_acc(s, zrow, acc)
            plsc.subcore_barrier()

            def chunk_body(j, carry, fb=fb):
                pltpu.async_copy(xp.at[fb].at[idx_s.at[j]], rows, sem).wait()
                pltpu.sync_copy(rows, acc.at[idx_d.at[j]], add=True)
                return carry

            lax.fori_loop(0, CHUNKS, chunk_body, 0)
            plsc.subcore_barrier()
            _write_acc(s, acc, out.at[c].at[fb])
            plsc.subcore_barrier()

    return seg


@functools.partial(
    pl.kernel,
    out_type=(jax.ShapeDtypeStruct((NSC, N, 128), F32),
              jax.ShapeDtypeStruct((NW, CHUNKS, K, 16), F32)),
    mesh=_sc_mesh,
    scratch_types=[
        pltpu.VMEM((1, K), jnp.int32),
        pltpu.VMEM((1, K), jnp.int32),
        pltpu.VMEM((K, 128), F32),
        pltpu.VMEM((K, 128), F32),
        pltpu.VMEM((K, 128), F32),
        pltpu.VMEM((K, 16), F32),
        pltpu.VMEM_SHARED((N, 128), F32),
        pltpu.SemaphoreType.DMA,
    ],
    name="sc_gat_den",
)
def _sc_gat_den(att, srcp, dstp, zrow, den_out, ee_out,
                idx_s, idx_d, alv, arcv, eev, eenarrow, acc, sem):
    c = lax.axis_index("c")
    s = lax.axis_index("s")
    wid = c * NTILE + s

    def zero_body(k, carry):
        for u in range(8):
            eev[k, pl.ds(u * 16, 16)] = jnp.zeros((16,), F32)
        return carry

    lax.fori_loop(0, K, zero_body, 0)
    _zero_acc(s, zrow, acc)
    plsc.subcore_barrier()

    def chunk_body(j, carry):
        pltpu.sync_copy(srcp.at[wid].at[j], idx_s)
        pltpu.sync_copy(dstp.at[wid].at[j], idx_d)
        pltpu.async_copy(att.at[idx_s.at[0]], alv, sem).wait()
        pltpu.async_copy(att.at[idx_d.at[0]], arcv, sem).wait()

        def edge_body(k, carry2):
            t = alv[k, pl.ds(0, 16)] + arcv[k, pl.ds(16, 16)]
            e = jnp.maximum(t, t * 0.2)
            ee = jnp.exp(e - arcv[k, pl.ds(32, 16)])
            eev[k, pl.ds(0, 16)] = ee
            eenarrow[k, :] = ee
            return carry2

        lax.fori_loop(0, K, edge_body, 0)
        pltpu.sync_copy(eev, acc.at[idx_d.at[0]], add=True)
        pltpu.sync_copy(eenarrow, ee_out.at[wid].at[j])
        return carry

    lax.fori_loop(0, CHUNKS, chunk_body, 0)
    plsc.subcore_barrier()
    _write_acc(s, acc, den_out.at[c])


@functools.partial(
    pl.kernel,
    out_type=jax.ShapeDtypeStruct((HEADS, NSC, N, 128), F32),
    mesh=_sc_mesh,
    scratch_types=[
        pltpu.VMEM((CHUNKS, K), jnp.int32),
        pltpu.VMEM((CHUNKS, K), jnp.int32),
        pltpu.VMEM((1, K), F32),
        pltpu.VMEM((K, 128), F32),
        pltpu.VMEM_SHARED((N, 128), F32),
        pltpu.SemaphoreType.DMA,
    ],
    name="sc_gat_num",
)
def _sc_gat_num(hg, srcp, dstp, eecols, zrow, out,
                idx_s, idx_d, eev, rows, acc, sem):
    c = lax.axis_index("c")
    s = lax.axis_index("s")
    wid = c * NTILE + s
    pltpu.sync_copy(srcp.at[wid], idx_s)
    pltpu.sync_copy(dstp.at[wid], idx_d)

    def head_body(hd, carry0):
        _zero_acc(s, zrow, acc)
        plsc.subcore_barrier()

        def chunk_body(j, carry):
            pltpu.async_copy(hg.at[hd].at[idx_s.at[j]], rows, sem).wait()
            pltpu.sync_copy(eecols.at[wid].at[hd].at[j], eev)
            for kk in range(K // 16):
                v = eev[0, pl.ds(kk * 16, 16)]
                for k16 in range(16):
                    f = jnp.full((16,), v[k16], F32)
                    k = kk * 16 + k16
                    for u in range(8):
                        rows[k, pl.ds(u * 16, 16)] = (
                            rows[k, pl.ds(u * 16, 16)] * f)
            pltpu.sync_copy(rows, acc.at[idx_d.at[j]], add=True)
            return carry

        lax.fori_loop(0, CHUNKS, chunk_body, 0)
        plsc.subcore_barrier()
        _write_acc(s, acc, out.at[hd].at[c])
        plsc.subcore_barrier()
        return carry0

    lax.fori_loop(0, HEADS, head_body, 0)


RB = 400
NBLK = N // RB


def _full(shape):
    return pl.BlockSpec(shape, lambda *_: (0,) * len(shape))


def _gin_mlp_body(nb, f2, xr, p, w1r, b1r, w2r, b2r, zr, statr, accr):
    i = pl.program_id(0)
    a = jnp.zeros((RB, f2), F32)
    for fb in range(nb):
        h0 = xr[fb] + p[0, fb] + p[1, fb]
        a = a + jnp.dot(h0, w1r[fb], preferred_element_type=F32)
    a = jnp.maximum(a + b1r[...], 0.0)
    z = jnp.dot(a, w2r[...], preferred_element_type=F32) + b2r[...]
    zr[...] = z

    @pl.when(i == 0)
    def _():
        accr[...] = jnp.zeros_like(accr)

    accr[0:1, :] += jnp.sum(z, axis=0, keepdims=True)
    accr[1:2, :] += jnp.sum(z * z, axis=0, keepdims=True)

    @pl.when(i == NBLK - 1)
    def _():
        statr[...] = accr[...]


def _gin_mlp(h, p, w1, b1, w2, b2):
    nb = h.shape[0]
    f2 = w2.shape[0]
    return pl.pallas_call(
        functools.partial(_gin_mlp_body, nb, f2),
        grid=(NBLK,),
        in_specs=[
            pl.BlockSpec((nb, RB, 128), lambda i: (0, i, 0)),
            pl.BlockSpec((NSC, nb, RB, 128), lambda i: (0, 0, i, 0)),
            _full((nb, 128, f2)),
            _full((1, f2)),
            _full((f2, f2)),
            _full((1, f2)),
        ],
        out_specs=[
            pl.BlockSpec((RB, f2), lambda i: (i, 0)),
            pl.BlockSpec((2, f2), lambda i: (0, 0)),
        ],
        out_shape=[
            jax.ShapeDtypeStruct((N, f2), F32),
            jax.ShapeDtypeStruct((2, f2), F32),
        ],
        scratch_shapes=[pltpu.VMEM((2, f2), F32)],
    )(h, p, w1.reshape(nb, 128, f2), b1.reshape(1, f2), w2, b2.reshape(1, f2))


def _bn_apply_body(nb2, zr, statr, gr, br, outr):
    m = statr[0:1, :] * (1.0 / N)
    v = statr[1:2, :] * (1.0 / N) - m * m
    hblk = jnp.maximum((zr[...] - m) * lax.rsqrt(v + 1e-5) * gr[...] + br[...],
                       0.0)
    outr[...] = jnp.transpose(hblk.reshape(RB, nb2, 128), (1, 0, 2))


def _bn_apply(z, stat, g, b):
    f2 = z.shape[1]
    nb2 = f2 // 128
    return pl.pallas_call(
        functools.partial(_bn_apply_body, nb2),
        grid=(NBLK,),
        in_specs=[
            pl.BlockSpec((RB, f2), lambda i: (i, 0)),
            _full((2, f2)),
            _full((1, f2)),
            _full((1, f2)),
        ],
        out_specs=pl.BlockSpec((nb2, RB, 128), lambda i: (0, i, 0)),
        out_shape=jax.ShapeDtypeStruct((nb2, N, 128), F32),
    )(z, stat, g.reshape(1, f2), b.reshape(1, f2))


def _gat_pre_body(hr, wr, asr, adr, hgr, alr, arr, malr, maccr):
    i = pl.program_id(0)
    hg = jnp.zeros((RB, HEADS * H), F32)
    for fb in range(2):
        hg = hg + jnp.dot(hr[fb], wr[fb], preferred_element_type=F32)
    hg3 = hg.reshape(RB, HEADS, H)
    al = jnp.sum(hg3 * asr[...][None, :, :], axis=-1)
    ar = jnp.sum(hg3 * adr[...][None, :, :], axis=-1)
    zpad = jnp.zeros((RB, 8), F32)
    alr[...] = jnp.concatenate([al, zpad], axis=1)
    arr[...] = jnp.concatenate([ar, zpad], axis=1)
    hgr[...] = jnp.transpose(hg3, (1, 0, 2))

    @pl.when(i == 0)
    def _():
        maccr[...] = jnp.full_like(maccr, -1e30)

    maccr[0:1, 0:8] = jnp.maximum(maccr[0:1, 0:8],
                                  jnp.max(al, axis=0, keepdims=True))

    @pl.when(i == NBLK - 1)
    def _():
        malr[0:1, 0:8] = maccr[0:1, 0:8]
        malr[0:1, 8:16] = jnp.zeros((1, 8), F32)


def _gat_pre(h, gatw, gatas, gatad):
    return pl.pallas_call(
        _gat_pre_body,
        grid=(NBLK,),
        in_specs=[
            pl.BlockSpec((2, RB, 128), lambda i: (0, i, 0)),
            _full((2, 128, HEADS * H)),
            _full((HEADS, H)),
            _full((HEADS, H)),
        ],
        out_specs=[
            pl.BlockSpec((HEADS, RB, 128), lambda i: (0, i, 0)),
            pl.BlockSpec((RB, 16), lambda i: (i, 0)),
            pl.BlockSpec((RB, 16), lambda i: (i, 0)),
            pl.BlockSpec((1, 16), lambda i: (0, 0)),
        ],
        out_shape=[
            jax.ShapeDtypeStruct((HEADS, N, 128), F32),
            jax.ShapeDtypeStruct((N, 16), F32),
            jax.ShapeDtypeStruct((N, 16), F32),
            jax.ShapeDtypeStruct((1, 16), F32),
        ],
        scratch_shapes=[pltpu.VMEM((1, 16), F32)],
    )(h, gatw.reshape(2, 128, HEADS * H), gatas, gatad)


def _gat_att_body(alr, arr, malr, outr):
    t = malr[...] + arr[...]
    c = jnp.maximum(t, 0.2 * t)
    outr[...] = jnp.concatenate(
        [alr[...], arr[...], c, jnp.zeros((RB, 128 - 48), F32)], axis=1)


def _gat_att(al, ar, mal):
    return pl.pallas_call(
        _gat_att_body,
        grid=(NBLK,),
        in_specs=[pl.BlockSpec((RB, 16), lambda i: (i, 0)),
                  pl.BlockSpec((RB, 16), lambda i: (i, 0)),
                  _full((1, 16))],
        out_specs=pl.BlockSpec((RB, 128), lambda i: (i, 0)),
        out_shape=jax.ShapeDtypeStruct((N, 128), F32),
    )(al, ar, mal)


def _gat_combine_body(alr, arr, malr, hgr, denr, numr, gbr, zr, statr, accr):
    i = pl.program_id(0)
    al8 = alr[:, 0:8]
    ar8 = arr[:, 0:8]
    mal8 = malr[0:1, 0:8]
    t = al8 + ar8
    tc = mal8 + ar8
    es = jnp.exp(jnp.maximum(t, 0.2 * t) - jnp.maximum(tc, 0.2 * tc))
    o = jnp.zeros((RB, 128), F32)
    for hd in range(HEADS):
        esh = es[:, hd:hd + 1]
        num_h = numr[hd, 0] + numr[hd, 1] + hgr[hd] * esh
        den_h = denr[0, :, hd:hd + 1] + denr[1, :, hd:hd + 1] + esh + 1e-16
        o = o + num_h / den_h
    z = o * (1.0 / HEADS) + gbr[...]
    zr[...] = z

    @pl.when(i == 0)
    def _():
        accr[...] = jnp.zeros_like(accr)

    accr[0:1, :] += jnp.sum(z, axis=0, keepdims=True)
    accr[1:2, :] += jnp.sum(z * z, axis=0, keepdims=True)

    @pl.when(i == NBLK - 1)
    def _():
        statr[...] = accr[...]


def _gat_combine(al, ar, mal, hgat, den, num, gatb):
    return pl.pallas_call(
        _gat_combine_body,
        grid=(NBLK,),
        in_specs=[
            pl.BlockSpec((RB, 16), lambda i: (i, 0)),
            pl.BlockSpec((RB, 16), lambda i: (i, 0)),
            _full((1, 16)),
            pl.BlockSpec((HEADS, RB, 128), lambda i: (0, i, 0)),
            pl.BlockSpec((NSC, RB, 128), lambda i: (0, i, 0)),
            pl.BlockSpec((HEADS, NSC, RB, 128), lambda i: (0, 0, i, 0)),
            _full((1, 128)),
        ],
        out_specs=[
            pl.BlockSpec((RB, 128), lambda i: (i, 0)),
            pl.BlockSpec((2, 128), lambda i: (0, 0)),
        ],
        out_shape=[
            jax.ShapeDtypeStruct((N, 128), F32),
            jax.ShapeDtypeStruct((2, 128), F32),
        ],
        scratch_shapes=[pltpu.VMEM((2, 128), F32)],
    )(al, ar, mal, hgat, den, num, gatb.reshape(1, 128))


def _pool_mlp_body(hr, br, pwr, pbr, f1wr, f1br, f2wr, f2br, outr):
    h5 = hr[...]
    gate = lax.dot_general(pwr[...], h5, (((1,), (1,)), ((), ()))) + pbr[0, 0]
    bt = br[...]
    gid = lax.broadcasted_iota(jnp.int32, (G, N), 0)
    oh = (gid == bt).astype(F32)
    gmax = jnp.max(jnp.where(oh > 0.0, gate, -1e30), axis=1, keepdims=True)
    mn = lax.dot_general(gmax, oh, (((0,), (0,)), ((), ())))
    ge = jnp.exp(gate - mn)
    dg = lax.dot_general(oh, ge, (((1,), (1,)), ((), ())))
    dn = lax.dot_general(dg, oh, (((0,), (0,)), ((), ())))
    alpha = ge / (dn + 1e-16)
    pooled = jnp.dot(oh * alpha, h5, preferred_element_type=F32)
    hh = jnp.maximum(jnp.dot(pooled, f1wr[...], preferred_element_type=F32)
                     + f1br[...], 0.0)
    outr[...] = jnp.dot(hh, f2wr[...], preferred_element_type=F32) + f2br[...]


def _pool_mlp(h5, batch, poolw, poolb, fc1w, fc1b, fc2w, fc2b):
    return pl.pallas_call(
        _pool_mlp_body,
        in_specs=[
            _full((N, 128)),
            _full((1, N)),
            _full((1, 128)),
            _full((1, 1)),
            _full((128, 128)),
            _full((1, 128)),
            _full((128, 1)),
            _full((1, 1)),
        ],
        out_specs=_full((G, 1)),
        out_shape=jax.ShapeDtypeStruct((G, 1), F32),
    )(h5, batch, poolw.reshape(1, 128), poolb.reshape(1, 1), fc1w,
      fc1b.reshape(1, 128), fc2w, fc2b.reshape(1, 1))


def kernel(x, edge_index, batch, g1w1, g1b1, g1w2, g1b2, bn1g, bn1b,
           g2w1, g2b1, g2w2, g2b2, bn2g, bn2b,
           g3w1, g3b1, g3w2, g3b2, bn3g, bn3b,
           g4w1, g4b1, g4w2, g4b2, bn4g, bn4b,
           gatw, gatas, gatad, gatb, bn5g, bn5b,
           poolw, poolb, fc1w, fc1b, fc2w, fc2b):
    src = edge_index[0].astype(jnp.int32).reshape(NW, CHUNKS, K)
    dst = edge_index[1].astype(jnp.int32).reshape(NW, CHUNKS, K)
    zrow = jnp.zeros((ROWS_A, 128), F32)

    h = x.reshape(1, N, 128)
    layers = [
        (g1w1, g1b1, g1w2, g1b2, bn1g, bn1b),
        (g2w1, g2b1, g2w2, g2b2, bn2g, bn2b),
        (g3w1, g3b1, g3w2, g3b2, bn3g, bn3b),
        (g4w1, g4b1, g4w2, g4b2, bn4g, bn4b),
    ]
    for (w1, b1, w2, b2, bg, bb) in layers:
        nb = h.shape[0]
        p = _make_sc_segsum(nb)(h, src, dst, zrow)
        z, stat = _gin_mlp(h, p, w1, b1, w2, b2)
        h = _bn_apply(z, stat, bg, bb)

    hgat, al, ar, mal = _gat_pre(h, gatw, gatas, gatad)
    att = _gat_att(al, ar, mal)
    den, ee = _sc_gat_den(att, src.reshape(NW, CHUNKS, 1, K),
                          dst.reshape(NW, CHUNKS, 1, K), zrow)
    eecols = jnp.moveaxis(ee, 3, 1)[:, 0:HEADS].reshape(NW, HEADS, CHUNKS, 1, K)
    num = _sc_gat_num(hgat, src, dst, eecols, zrow)
    z5, stat5 = _gat_combine(al, ar, mal, hgat, den, num, gatb)
    h5 = _bn_apply(z5, stat5, bn5g, bn5b)

    return _pool_mlp(h5.reshape(N, 128), batch.astype(jnp.int32).reshape(1, N),
                     poolw, poolb, fc1w, fc1b, fc2w, fc2b)


# speedup vs baseline: 15.8981x; 1.6378x over previous
import functools

import jax
import jax.numpy as jnp
from jax import lax
from jax.experimental import pallas as pl
from jax.experimental.pallas import tpu as pltpu
from jax.experimental.pallas import tpu_sc as plsc

N, E, G = 10000, 320000, 64
D, H, HEADS = 128, 128, 8
F32 = jnp.float32

NSC, NTILE = 2, 16
NW = NSC * NTILE
K = 80
CHUNKS = E // (NW * K)
ROWS_A = 624
TAIL0, TAILN = NTILE * ROWS_A, N - NTILE * ROWS_A

_sc_mesh = plsc.VectorSubcoreMesh(core_axis_name="c", subcore_axis_name="s")


def _zero_acc(s, zrow, acc):
    pltpu.sync_copy(zrow, acc.at[pl.ds(s * ROWS_A, ROWS_A)])

    @pl.when(s == NTILE - 1)
    def _():
        pltpu.sync_copy(zrow.at[pl.ds(0, TAILN)], acc.at[pl.ds(TAIL0, TAILN)])


def _write_acc(s, acc, out_ref):
    pltpu.sync_copy(acc.at[pl.ds(s * ROWS_A, ROWS_A)],
                    out_ref.at[pl.ds(s * ROWS_A, ROWS_A)])

    @pl.when(s == NTILE - 1)
    def _():
        pltpu.sync_copy(acc.at[pl.ds(TAIL0, TAILN)],
                        out_ref.at[pl.ds(TAIL0, TAILN)])


def _make_sc_segsum(nb):
    @functools.partial(
        pl.kernel,
        out_type=jax.ShapeDtypeStruct((NSC, nb, N, 128), F32),
        mesh=_sc_mesh,
        scratch_types=[
            pltpu.VMEM((CHUNKS, K), jnp.int32),
            pltpu.VMEM((1, K), jnp.int32),
            pltpu.VMEM((1, K), jnp.int32),
            pltpu.VMEM((K, 128), F32),
            pltpu.VMEM((K, 128), F32),
            pltpu.VMEM_SHARED((N, 128), F32),
            pltpu.SemaphoreType.DMA,
            pltpu.SemaphoreType.DMA,
            pltpu.SemaphoreType.DMA,
            pltpu.SemaphoreType.DMA,
        ],
        name=f"sc_segsum_nb{nb}",
    )
    def seg(xp, srcp, dstp, zrow, out, idx_s, idxd0, idxd1, rows0, rows1,
            acc, semg0, semg1, semd0, semd1):
        c = lax.axis_index("c")
        s = lax.axis_index("s")
        wid = c * NTILE + s
        pltpu.sync_copy(srcp.at[wid], idx_s)
        for fb in range(nb):
            _zero_acc(s, zrow, acc)
            plsc.subcore_barrier()
            pltpu.async_copy(xp.at[fb].at[idx_s.at[0]], rows0, semg0)
            pltpu.async_copy(dstp.at[wid].at[0], idxd0, semd0)

            def pair_body(t, carry, fb=fb):
                j0 = 2 * t
                pltpu.async_copy(xp.at[fb].at[idx_s.at[j0 + 1]], rows1, semg1)
                pltpu.async_copy(dstp.at[wid].at[j0 + 1], idxd1, semd1)
                pltpu.make_async_copy(xp.at[fb].at[idx_s.at[j0]], rows0,
                                      semg0).wait()
                pltpu.make_async_copy(dstp.at[wid].at[j0], idxd0,
                                      semd0).wait()
                pltpu.sync_copy(rows0, acc.at[idxd0.at[0]], add=True)
                pltpu.async_copy(xp.at[fb].at[idx_s.at[j0 + 2]], rows0, semg0)
                pltpu.async_copy(dstp.at[wid].at[j0 + 2], idxd0, semd0)
                pltpu.make_async_copy(xp.at[fb].at[idx_s.at[j0 + 1]], rows1,
                                      semg1).wait()
                pltpu.make_async_copy(dstp.at[wid].at[j0 + 1], idxd1,
                                      semd1).wait()
                pltpu.sync_copy(rows1, acc.at[idxd1.at[0]], add=True)
                return carry

            lax.fori_loop(0, CHUNKS // 2, pair_body, 0)
            pltpu.make_async_copy(xp.at[fb].at[idx_s.at[CHUNKS - 1]], rows0,
                                  semg0).wait()
            pltpu.make_async_copy(dstp.at[wid].at[CHUNKS - 1], idxd0,
                                  semd0).wait()
            pltpu.sync_copy(rows0, acc.at[idxd0.at[0]], add=True)
            plsc.subcore_barrier()
            _write_acc(s, acc, out.at[c].at[fb])
            plsc.subcore_barrier()

    return seg


@functools.partial(
    pl.kernel,
    out_type=(jax.ShapeDtypeStruct((NSC, N, 128), F32),
              jax.ShapeDtypeStruct((NW, CHUNKS, K, 16), F32)),
    mesh=_sc_mesh,
    scratch_types=[
        pltpu.VMEM((1, K), jnp.int32),
        pltpu.VMEM((1, K), jnp.int32),
        pltpu.VMEM((K, 128), F32),
        pltpu.VMEM((K, 128), F32),
        pltpu.VMEM((K, 128), F32),
        pltpu.VMEM((K, 16), F32),
        pltpu.VMEM_SHARED((N, 128), F32),
        pltpu.SemaphoreType.DMA,
    ],
    name="sc_gat_den",
)
def _sc_gat_den(att, srcp, dstp, zrow, den_out, ee_out,
                idx_s, idx_d, alv, arcv, eev, eenarrow, acc, sem):
    c = lax.axis_index("c")
    s = lax.axis_index("s")
    wid = c * NTILE + s

    def zero_body(k, carry):
        for u in range(8):
            eev[k, pl.ds(u * 16, 16)] = jnp.zeros((16,), F32)
        return carry

    lax.fori_loop(0, K, zero_body, 0)
    _zero_acc(s, zrow, acc)
    plsc.subcore_barrier()

    def chunk_body(j, carry):
        pltpu.sync_copy(srcp.at[wid].at[j], idx_s)
        pltpu.sync_copy(dstp.at[wid].at[j], idx_d)
        pltpu.async_copy(att.at[idx_s.at[0]], alv, sem).wait()
        pltpu.async_copy(att.at[idx_d.at[0]], arcv, sem).wait()

        def edge_body(k, carry2):
            t = alv[k, pl.ds(0, 16)] + arcv[k, pl.ds(16, 16)]
            e = jnp.maximum(t, t * 0.2)
            ee = jnp.exp(e - arcv[k, pl.ds(32, 16)])
            eev[k, pl.ds(0, 16)] = ee
            eenarrow[k, :] = ee
            return carry2

        lax.fori_loop(0, K, edge_body, 0)
        pltpu.sync_copy(eev, acc.at[idx_d.at[0]], add=True)
        pltpu.sync_copy(eenarrow, ee_out.at[wid].at[j])
        return carry

    lax.fori_loop(0, CHUNKS, chunk_body, 0)
    plsc.subcore_barrier()
    _write_acc(s, acc, den_out.at[c])


@functools.partial(
    pl.kernel,
    out_type=jax.ShapeDtypeStruct((HEADS, NSC, N, 128), F32),
    mesh=_sc_mesh,
    scratch_types=[
        pltpu.VMEM((CHUNKS, K), jnp.int32),
        pltpu.VMEM((1, K), jnp.int32),
        pltpu.VMEM((1, K), jnp.int32),
        pltpu.VMEM((1, K), F32),
        pltpu.VMEM((1, K), F32),
        pltpu.VMEM((K, 128), F32),
        pltpu.VMEM((K, 128), F32),
        pltpu.VMEM_SHARED((N, 128), F32),
        pltpu.SemaphoreType.DMA,
        pltpu.SemaphoreType.DMA,
        pltpu.SemaphoreType.DMA,
        pltpu.SemaphoreType.DMA,
        pltpu.SemaphoreType.DMA,
        pltpu.SemaphoreType.DMA,
    ],
    name="sc_gat_num",
)
def _sc_gat_num(hg, srcp, dstp, eecols, zrow, out,
                idx_s, idxd0, idxd1, eev0, eev1, rows0, rows1, acc,
                semg0, semg1, semd0, semd1, seme0, seme1):
    c = lax.axis_index("c")
    s = lax.axis_index("s")
    wid = c * NTILE + s
    pltpu.sync_copy(srcp.at[wid], idx_s)

    def scale(rows_ref, eev_ref):
        def grp(kk, carry):
            v = eev_ref[0, pl.ds(kk * 16, 16)]
            for k16 in range(16):
                f = jnp.full((16,), v[k16], F32)
                k = kk * 16 + k16
                for u in range(8):
                    rows_ref[k, pl.ds(u * 16, 16)] = (
                        rows_ref[k, pl.ds(u * 16, 16)] * f)
            return carry

        lax.fori_loop(0, K // 16, grp, 0)

    def head_body(hd, carry0):
        _zero_acc(s, zrow, acc)
        plsc.subcore_barrier()
        pltpu.async_copy(hg.at[hd].at[idx_s.at[0]], rows0, semg0)
        pltpu.async_copy(dstp.at[wid].at[0], idxd0, semd0)
        pltpu.async_copy(eecols.at[wid].at[hd].at[0], eev0, seme0)

        def pair_body(t, carry, hd=hd):
            j0 = 2 * t
            pltpu.async_copy(hg.at[hd].at[idx_s.at[j0 + 1]], rows1, semg1)
            pltpu.async_copy(dstp.at[wid].at[j0 + 1], idxd1, semd1)
            pltpu.async_copy(eecols.at[wid].at[hd].at[j0 + 1], eev1, seme1)
            pltpu.make_async_copy(hg.at[hd].at[idx_s.at[j0]], rows0,
                                  semg0).wait()
            pltpu.make_async_copy(dstp.at[wid].at[j0], idxd0, semd0).wait()
            pltpu.make_async_copy(eecols.at[wid].at[hd].at[j0], eev0,
                                  seme0).wait()
            scale(rows0, eev0)
            pltpu.sync_copy(rows0, acc.at[idxd0.at[0]], add=True)
            pltpu.async_copy(hg.at[hd].at[idx_s.at[j0 + 2]], rows0, semg0)
            pltpu.async_copy(dstp.at[wid].at[j0 + 2], idxd0, semd0)
            pltpu.async_copy(eecols.at[wid].at[hd].at[j0 + 2], eev0, seme0)
            pltpu.make_async_copy(hg.at[hd].at[idx_s.at[j0 + 1]], rows1,
                                  semg1).wait()
            pltpu.make_async_copy(dstp.at[wid].at[j0 + 1], idxd1,
                                  semd1).wait()
            pltpu.make_async_copy(eecols.at[wid].at[hd].at[j0 + 1], eev1,
                                  seme1).wait()
            scale(rows1, eev1)
            pltpu.sync_copy(rows1, acc.at[idxd1.at[0]], add=True)
            return carry

        lax.fori_loop(0, CHUNKS // 2, pair_body, 0)
        pltpu.make_async_copy(hg.at[hd].at[idx_s.at[CHUNKS - 1]], rows0,
                              semg0).wait()
        pltpu.make_async_copy(dstp.at[wid].at[CHUNKS - 1], idxd0,
                              semd0).wait()
        pltpu.make_async_copy(eecols.at[wid].at[hd].at[CHUNKS - 1], eev0,
                              seme0).wait()
        scale(rows0, eev0)
        pltpu.sync_copy(rows0, acc.at[idxd0.at[0]], add=True)
        plsc.subcore_barrier()
        _write_acc(s, acc, out.at[hd].at[c])
        plsc.subcore_barrier()
        return carry0

    lax.fori_loop(0, HEADS, head_body, 0)


RB = 400
NBLK = N // RB


def _full(shape):
    return pl.BlockSpec(shape, lambda *_: (0,) * len(shape))


def _gin_mlp_body(nb, f2, xr, p, w1r, b1r, w2r, b2r, zr, statr, accr):
    i = pl.program_id(0)
    a = jnp.zeros((RB, f2), F32)
    for fb in range(nb):
        h0 = xr[fb] + p[0, fb] + p[1, fb]
        a = a + jnp.dot(h0, w1r[fb], preferred_element_type=F32)
    a = jnp.maximum(a + b1r[...], 0.0)
    z = jnp.dot(a, w2r[...], preferred_element_type=F32) + b2r[...]
    zr[...] = z

    @pl.when(i == 0)
    def _():
        accr[...] = jnp.zeros_like(accr)

    accr[0:1, :] += jnp.sum(z, axis=0, keepdims=True)
    accr[1:2, :] += jnp.sum(z * z, axis=0, keepdims=True)

    @pl.when(i == NBLK - 1)
    def _():
        statr[...] = accr[...]


def _gin_mlp(h, p, w1, b1, w2, b2):
    nb = h.shape[0]
    f2 = w2.shape[0]
    return pl.pallas_call(
        functools.partial(_gin_mlp_body, nb, f2),
        grid=(NBLK,),
        in_specs=[
            pl.BlockSpec((nb, RB, 128), lambda i: (0, i, 0)),
            pl.BlockSpec((NSC, nb, RB, 128), lambda i: (0, 0, i, 0)),
            _full((nb, 128, f2)),
            _full((1, f2)),
            _full((f2, f2)),
            _full((1, f2)),
        ],
        out_specs=[
            pl.BlockSpec((RB, f2), lambda i: (i, 0)),
            pl.BlockSpec((2, f2), lambda i: (0, 0)),
        ],
        out_shape=[
            jax.ShapeDtypeStruct((N, f2), F32),
            jax.ShapeDtypeStruct((2, f2), F32),
        ],
        scratch_shapes=[pltpu.VMEM((2, f2), F32)],
    )(h, p, w1.reshape(nb, 128, f2), b1.reshape(1, f2), w2, b2.reshape(1, f2))


def _bn_apply_body(nb2, zr, statr, gr, br, outr):
    m = statr[0:1, :] * (1.0 / N)
    v = statr[1:2, :] * (1.0 / N) - m * m
    hblk = jnp.maximum((zr[...] - m) * lax.rsqrt(v + 1e-5) * gr[...] + br[...],
                       0.0)
    outr[...] = jnp.transpose(hblk.reshape(RB, nb2, 128), (1, 0, 2))


def _bn_apply(z, stat, g, b):
    f2 = z.shape[1]
    nb2 = f2 // 128
    return pl.pallas_call(
        functools.partial(_bn_apply_body, nb2),
        grid=(NBLK,),
        in_specs=[
            pl.BlockSpec((RB, f2), lambda i: (i, 0)),
            _full((2, f2)),
            _full((1, f2)),
            _full((1, f2)),
        ],
        out_specs=pl.BlockSpec((nb2, RB, 128), lambda i: (0, i, 0)),
        out_shape=jax.ShapeDtypeStruct((nb2, N, 128), F32),
    )(z, stat, g.reshape(1, f2), b.reshape(1, f2))


def _gat_pre_body(hr, wr, asr, adr, hgr, alr, arr, malr, maccr):
    i = pl.program_id(0)
    hg = jnp.zeros((RB, HEADS * H), F32)
    for fb in range(2):
        hg = hg + jnp.dot(hr[fb], wr[fb], preferred_element_type=F32)
    hg3 = hg.reshape(RB, HEADS, H)
    al = jnp.sum(hg3 * asr[...][None, :, :], axis=-1)
    ar = jnp.sum(hg3 * adr[...][None, :, :], axis=-1)
    zpad = jnp.zeros((RB, 8), F32)
    alr[...] = jnp.concatenate([al, zpad], axis=1)
    arr[...] = jnp.concatenate([ar, zpad], axis=1)
    hgr[...] = jnp.transpose(hg3, (1, 0, 2))

    @pl.when(i == 0)
    def _():
        maccr[...] = jnp.full_like(maccr, -1e30)

    maccr[0:1, 0:8] = jnp.maximum(maccr[0:1, 0:8],
                                  jnp.max(al, axis=0, keepdims=True))

    @pl.when(i == NBLK - 1)
    def _():
        malr[0:1, 0:8] = maccr[0:1, 0:8]
        malr[0:1, 8:16] = jnp.zeros((1, 8), F32)


def _gat_pre(h, gatw, gatas, gatad):
    return pl.pallas_call(
        _gat_pre_body,
        grid=(NBLK,),
        in_specs=[
            pl.BlockSpec((2, RB, 128), lambda i: (0, i, 0)),
            _full((2, 128, HEADS * H)),
            _full((HEADS, H)),
            _full((HEADS, H)),
        ],
        out_specs=[
            pl.BlockSpec((HEADS, RB, 128), lambda i: (0, i, 0)),
            pl.BlockSpec((RB, 16), lambda i: (i, 0)),
            pl.BlockSpec((RB, 16), lambda i: (i, 0)),
            pl.BlockSpec((1, 16), lambda i: (0, 0)),
        ],
        out_shape=[
            jax.ShapeDtypeStruct((HEADS, N, 128), F32),
            jax.ShapeDtypeStruct((N, 16), F32),
            jax.ShapeDtypeStruct((N, 16), F32),
            jax.ShapeDtypeStruct((1, 16), F32),
        ],
        scratch_shapes=[pltpu.VMEM((1, 16), F32)],
    )(h, gatw.reshape(2, 128, HEADS * H), gatas, gatad)


def _gat_att_body(alr, arr, malr, outr):
    t = malr[...] + arr[...]
    c = jnp.maximum(t, 0.2 * t)
    outr[...] = jnp.concatenate(
        [alr[...], arr[...], c, jnp.zeros((RB, 128 - 48), F32)], axis=1)


def _gat_att(al, ar, mal):
    return pl.pallas_call(
        _gat_att_body,
        grid=(NBLK,),
        in_specs=[pl.BlockSpec((RB, 16), lambda i: (i, 0)),
                  pl.BlockSpec((RB, 16), lambda i: (i, 0)),
                  _full((1, 16))],
        out_specs=pl.BlockSpec((RB, 128), lambda i: (i, 0)),
        out_shape=jax.ShapeDtypeStruct((N, 128), F32),
    )(al, ar, mal)


def _gat_combine_body(alr, arr, malr, hgr, denr, numr, gbr, zr, statr, accr):
    i = pl.program_id(0)
    al8 = alr[:, 0:8]
    ar8 = arr[:, 0:8]
    mal8 = malr[0:1, 0:8]
    t = al8 + ar8
    tc = mal8 + ar8
    es = jnp.exp(jnp.maximum(t, 0.2 * t) - jnp.maximum(tc, 0.2 * tc))
    o = jnp.zeros((RB, 128), F32)
    for hd in range(HEADS):
        esh = es[:, hd:hd + 1]
        num_h = numr[hd, 0] + numr[hd, 1] + hgr[hd] * esh
        den_h = denr[0, :, hd:hd + 1] + denr[1, :, hd:hd + 1] + esh + 1e-16
        o = o + num_h / den_h
    z = o * (1.0 / HEADS) + gbr[...]
    zr[...] = z

    @pl.when(i == 0)
    def _():
        accr[...] = jnp.zeros_like(accr)

    accr[0:1, :] += jnp.sum(z, axis=0, keepdims=True)
    accr[1:2, :] += jnp.sum(z * z, axis=0, keepdims=True)

    @pl.when(i == NBLK - 1)
    def _():
        statr[...] = accr[...]


def _gat_combine(al, ar, mal, hgat, den, num, gatb):
    return pl.pallas_call(
        _gat_combine_body,
        grid=(NBLK,),
        in_specs=[
            pl.BlockSpec((RB, 16), lambda i: (i, 0)),
            pl.BlockSpec((RB, 16), lambda i: (i, 0)),
            _full((1, 16)),
            pl.BlockSpec((HEADS, RB, 128), lambda i: (0, i, 0)),
            pl.BlockSpec((NSC, RB, 128), lambda i: (0, i, 0)),
            pl.BlockSpec((HEADS, NSC, RB, 128), lambda i: (0, 0, i, 0)),
            _full((1, 128)),
        ],
        out_specs=[
            pl.BlockSpec((RB, 128), lambda i: (i, 0)),
            pl.BlockSpec((2, 128), lambda i: (0, 0)),
        ],
        out_shape=[
            jax.ShapeDtypeStruct((N, 128), F32),
            jax.ShapeDtypeStruct((2, 128), F32),
        ],
        scratch_shapes=[pltpu.VMEM((2, 128), F32)],
    )(al, ar, mal, hgat, den, num, gatb.reshape(1, 128))


def _pool_mlp_body(hr, br, pwr, pbr, f1wr, f1br, f2wr, f2br, outr):
    h5 = hr[...]
    gate = lax.dot_general(pwr[...], h5, (((1,), (1,)), ((), ()))) + pbr[0, 0]
    bt = br[...]
    gid = lax.broadcasted_iota(jnp.int32, (G, N), 0)
    oh = (gid == bt).astype(F32)
    gmax = jnp.max(jnp.where(oh > 0.0, gate, -1e30), axis=1, keepdims=True)
    mn = lax.dot_general(gmax, oh, (((0,), (0,)), ((), ())))
    ge = jnp.exp(gate - mn)
    dg = lax.dot_general(oh, ge, (((1,), (1,)), ((), ())))
    dn = lax.dot_general(dg, oh, (((0,), (0,)), ((), ())))
    alpha = ge / (dn + 1e-16)
    pooled = jnp.dot(oh * alpha, h5, preferred_element_type=F32)
    hh = jnp.maximum(jnp.dot(pooled, f1wr[...], preferred_element_type=F32)
                     + f1br[...], 0.0)
    outr[...] = jnp.dot(hh, f2wr[...], preferred_element_type=F32) + f2br[...]


def _pool_mlp(h5, batch, poolw, poolb, fc1w, fc1b, fc2w, fc2b):
    return pl.pallas_call(
        _pool_mlp_body,
        in_specs=[
            _full((N, 128)),
            _full((1, N)),
            _full((1, 128)),
            _full((1, 1)),
            _full((128, 128)),
            _full((1, 128)),
            _full((128, 1)),
            _full((1, 1)),
        ],
        out_specs=_full((G, 1)),
        out_shape=jax.ShapeDtypeStruct((G, 1), F32),
    )(h5, batch, poolw.reshape(1, 128), poolb.reshape(1, 1), fc1w,
      fc1b.reshape(1, 128), fc2w, fc2b.reshape(1, 1))


def kernel(x, edge_index, batch, g1w1, g1b1, g1w2, g1b2, bn1g, bn1b,
           g2w1, g2b1, g2w2, g2b2, bn2g, bn2b,
           g3w1, g3b1, g3w2, g3b2, bn3g, bn3b,
           g4w1, g4b1, g4w2, g4b2, bn4g, bn4b,
           gatw, gatas, gatad, gatb, bn5g, bn5b,
           poolw, poolb, fc1w, fc1b, fc2w, fc2b):
    src = edge_index[0].astype(jnp.int32).reshape(NW, CHUNKS, K)
    dst4 = edge_index[1].astype(jnp.int32).reshape(NW, CHUNKS, 1, K)
    zrow = jnp.zeros((ROWS_A, 128), F32)

    h = x.reshape(1, N, 128)
    layers = [
        (g1w1, g1b1, g1w2, g1b2, bn1g, bn1b),
        (g2w1, g2b1, g2w2, g2b2, bn2g, bn2b),
        (g3w1, g3b1, g3w2, g3b2, bn3g, bn3b),
        (g4w1, g4b1, g4w2, g4b2, bn4g, bn4b),
    ]
    for (w1, b1, w2, b2, bg, bb) in layers:
        nb = h.shape[0]
        p = _make_sc_segsum(nb)(h, src, dst4, zrow)
        z, stat = _gin_mlp(h, p, w1, b1, w2, b2)
        h = _bn_apply(z, stat, bg, bb)

    hgat, al, ar, mal = _gat_pre(h, gatw, gatas, gatad)
    att = _gat_att(al, ar, mal)
    den, ee = _sc_gat_den(att, src.reshape(NW, CHUNKS, 1, K), dst4, zrow)
    eecols = jnp.moveaxis(ee, 3, 1)[:, 0:HEADS].reshape(NW, HEADS, CHUNKS, 1, K)
    num = _sc_gat_num(hgat, src, dst4, eecols, zrow)
    z5, stat5 = _gat_combine(al, ar, mal, hgat, den, num, gatb)
    h5 = _bn_apply(z5, stat5, bn5g, bn5b)

    return _pool_mlp(h5.reshape(N, 128), batch.astype(jnp.int32).reshape(1, N),
                     poolw, poolb, fc1w, fc1b, fc2w, fc2b)
